# baseline (speedup 1.0000x reference)
import functools

import jax
import jax.numpy as jnp
from jax import lax
from jax.experimental import pallas as pl
from jax.experimental.pallas import tpu as pltpu
from jax.experimental.pallas import tpu_sc as plsc

NUM_CORES = 2
NUM_SUBCORES = 16
NUM_WORKERS = NUM_CORES * NUM_SUBCORES
LANES = 16
CHUNK = 128
FACTORS = 64


def _sc_body(users_hbm, items_hbm, x_hbm, y_hbm, out_hbm,
             uidx, iidx, urows, irows, out_v, sem):
    wid = lax.axis_index("s") * NUM_CORES + lax.axis_index("c")
    n_chunks = uidx.shape[0]
    base = wid * n_chunks

    pltpu.sync_copy(users_hbm.at[pl.ds(base, n_chunks)], uidx)
    pltpu.sync_copy(items_hbm.at[pl.ds(base, n_chunks)], iidx)

    copies = []
    for j in range(n_chunks):
        copies.append(pltpu.async_copy(x_hbm.at[uidx.at[j]], urows.at[j], sem))
        copies.append(pltpu.async_copy(y_hbm.at[iidx.at[j]], irows.at[j], sem))
    for c in copies:
        c.wait()

    iota = lax.broadcasted_iota(jnp.int32, (LANES,), 0)
    rot_idx = [((iota + k) & (LANES - 1)).reshape(LANES, 1)
               for k in (8, 4, 2, 1)]
    _gdn = lax.GatherDimensionNumbers(offset_dims=(), collapsed_slice_dims=(0,),
                                      start_index_map=(0,))

    def _rot(v, ridx):
        return lax.gather(v, ridx, _gdn, slice_sizes=(1,),
                          mode=lax.GatherScatterMode.PROMISE_IN_BOUNDS)

    for j in range(n_chunks):
        def group(g, _, j=j):
            acc = jnp.zeros((LANES,), jnp.float32)
            for l in range(LANES):
                r = g * LANES + l
                p = urows[j, r, pl.ds(0, LANES)] * irows[j, r, pl.ds(0, LANES)]
                for t in range(1, FACTORS // LANES):
                    p = p + (urows[j, r, pl.ds(t * LANES, LANES)]
                             * irows[j, r, pl.ds(t * LANES, LANES)])
                for ridx in rot_idx:
                    p = p + _rot(p, ridx)
                acc = jnp.where(iota == l, p, acc)
            out_v[j, pl.ds(g * LANES, LANES)] = acc
            return 0
        lax.fori_loop(0, CHUNK // LANES, group, 0)

    pltpu.sync_copy(out_v, out_hbm.at[pl.ds(base, n_chunks)])


def _build(n_rows, n_chunks):
    mesh = plsc.VectorSubcoreMesh(core_axis_name="c", subcore_axis_name="s",
                                  num_cores=NUM_CORES,
                                  num_subcores=NUM_SUBCORES)
    return pl.kernel(
        _sc_body,
        out_type=jax.ShapeDtypeStruct((n_rows, CHUNK), jnp.float32),
        mesh=mesh,
        scratch_types=[
            pltpu.VMEM((n_chunks, CHUNK), jnp.int32),
            pltpu.VMEM((n_chunks, CHUNK), jnp.int32),
            pltpu.VMEM((n_chunks, CHUNK, FACTORS), jnp.float32),
            pltpu.VMEM((n_chunks, CHUNK, FACTORS), jnp.float32),
            pltpu.VMEM((n_chunks, CHUNK), jnp.float32),
            pltpu.SemaphoreType.DMA,
        ],
        compiler_params=pltpu.CompilerParams(use_tc_tiling_on_sc=False),
    )


def kernel(users, items, x, y):
    batch = users.shape[0]
    n_rows = batch // CHUNK
    n_chunks = n_rows // NUM_WORKERS
    u2 = users.astype(jnp.int32).reshape(n_rows, CHUNK)
    i2 = items.astype(jnp.int32).reshape(n_rows, CHUNK)
    out2 = _build(n_rows, n_chunks)(u2, i2, x, y)
    return out2.reshape(batch)
